# baseline (speedup 1.0000x reference)
import jax
import jax.numpy as jnp
from jax import lax
from jax.experimental import pallas as pl
from jax.experimental.pallas import tpu as pltpu
from jax.experimental.pallas import tpu_sc as plsc

B, S = 16384, 200
V, D = 50, 16
N = B * S
NW = 32
L = 16
NTILES = S * (D // 8) * (B // 128)
PER_W = NTILES // NW
TPC = 32
NCHUNKS = PER_W // TPC
CIDX = TPC * 128
GROUPS = CIDX // L
COUT = TPC * 1024


def _lookup_body(xt_hbm, emb_hbm, out_hbm, emb_v, idx_v, rows_v, sem_idx, sem_st):
    cid = lax.axis_index("c")
    sid = lax.axis_index("s")
    wid = sid * 2 + cid
    t_base = wid * PER_W

    pltpu.sync_copy(emb_hbm, emb_v)

    def start_idx(c, b):
        t0 = t_base + c * TPC
        s = t0 >> 8
        nt0 = t0 & 127
        pltpu.async_copy(
            xt_hbm.at[pl.ds(s * B + nt0 * 128, CIDX)], idx_v.at[b], sem_idx.at[b]
        )

    start_idx(0, 0)
    start_idx(1, 1)

    lane = lax.iota(jnp.int32, L)

    @pl.loop(0, NCHUNKS, step=2)
    def _super(g0):
        for b in range(2):
            c = g0 + b
            t0 = t_base + c * TPC
            dt = (t0 >> 7) & 1

            @pl.when(c >= 2)
            def _():
                pltpu.make_async_copy(
                    rows_v.at[b], out_hbm.at[pl.ds(0, COUT)], sem_st.at[b]
                ).wait()

            pltpu.make_async_copy(
                xt_hbm.at[pl.ds(0, CIDX)], idx_v.at[b], sem_idx.at[b]
            ).wait()

            idx_ref = idx_v.at[b]
            rows_ref = rows_v.at[b]

            @plsc.parallel_loop(0, GROUPS, unroll=8)
            def _grp(g):
                idxv = idx_ref[pl.ds(g * L, L)]
                gbase = idxv * D + dt * 8
                sbase = lane + ((g >> 3) << 10) + ((g & 7) << 4)
                for di in range(8):
                    vals = plsc.load_gather(emb_v, [gbase + di])
                    plsc.store_scatter(rows_ref, [sbase + (di << 7)], vals)

            @pl.when(c + 2 < NCHUNKS)
            def _():
                start_idx(c + 2, b)

            pltpu.async_copy(
                rows_v.at[b],
                out_hbm.at[pl.ds(t0 * 1024, COUT)],
                sem_st.at[b],
            )

    for b in range(2):
        pltpu.make_async_copy(
            rows_v.at[b], out_hbm.at[pl.ds(0, COUT)], sem_st.at[b]
        ).wait()


@jax.jit
def _lookup(xt_flat, emb_flat):
    mesh = plsc.VectorSubcoreMesh(core_axis_name="c", subcore_axis_name="s")
    return pl.kernel(
        _lookup_body,
        out_type=jax.ShapeDtypeStruct((N * D,), jnp.float32),
        mesh=mesh,
        scratch_types=[
            pltpu.VMEM((V * D,), jnp.float32),
            pltpu.VMEM((2, CIDX), jnp.int32),
            pltpu.VMEM((2, COUT), jnp.float32),
            pltpu.SemaphoreType.DMA((2,)),
            pltpu.SemaphoreType.DMA((2,)),
        ],
        compiler_params=pltpu.CompilerParams(
            use_tc_tiling_on_sc=False, needs_layout_passes=False
        ),
    )(xt_flat, emb_flat)


def kernel(x, embedding):
    xt_flat = jnp.transpose(x).reshape(N).astype(jnp.int32)
    out = _lookup(xt_flat, embedding.reshape(V * D))
    out5 = out.reshape(S, D // 8, B // 128, 8, 128)
    return jnp.transpose(out5, (2, 4, 0, 1, 3)).reshape(B, S, D)


# speedup vs baseline: 1.2581x; 1.2581x over previous
import jax
import jax.numpy as jnp
from jax import lax
from jax.experimental import pallas as pl
from jax.experimental.pallas import tpu as pltpu
from jax.experimental.pallas import tpu_sc as plsc

B, S = 16384, 200
V, D = 50, 16
N = B * S
NW = 32
L = 16
NTILES = S * (D // 8) * (B // 128)
PER_W = NTILES // NW
TPC = 32
NCHUNKS = PER_W // TPC
CIDX = TPC * 128
GROUPS = CIDX // L
COUT = TPC * 1024


def _lookup_body(xt_hbm, emb_hbm, out_hbm, emb_v, idx_v, rows_v, sem_idx, sem_st):
    cid = lax.axis_index("c")
    sid = lax.axis_index("s")
    wid = sid * 2 + cid
    t_base = wid * PER_W

    pltpu.sync_copy(emb_hbm, emb_v)

    def start_idx(c, b):
        t0 = t_base + c * TPC
        s = t0 >> 8
        nt0 = t0 & 127
        pltpu.async_copy(
            xt_hbm.at[pl.ds(s * B + nt0 * 128, CIDX)], idx_v.at[b], sem_idx.at[b]
        )

    start_idx(0, 0)
    start_idx(1, 1)

    lane = lax.iota(jnp.int32, L)

    @pl.loop(0, NCHUNKS, step=2)
    def _super(g0):
        for b in range(2):
            c = g0 + b
            t0 = t_base + c * TPC
            dt = (t0 >> 7) & 1

            @pl.when(c >= 2)
            def _():
                pltpu.make_async_copy(
                    rows_v.at[b], out_hbm.at[pl.ds(0, COUT)], sem_st.at[b]
                ).wait()

            pltpu.make_async_copy(
                xt_hbm.at[pl.ds(0, CIDX)], idx_v.at[b], sem_idx.at[b]
            ).wait()

            idx_ref = idx_v.at[b]
            rows_ref = rows_v.at[b]

            @plsc.parallel_loop(0, GROUPS, unroll=2)
            def _grp(g):
                idxv = idx_ref[pl.ds(g * L, L)]
                gbase = idxv * D + dt * 8
                sbase = lane + ((g >> 3) << 10) + ((g & 7) << 4)
                for di in range(8):
                    vals = plsc.load_gather(emb_v, [gbase + di])
                    plsc.store_scatter(rows_ref, [sbase + (di << 7)], vals)

            @pl.when(c + 2 < NCHUNKS)
            def _():
                start_idx(c + 2, b)

            pltpu.async_copy(
                rows_v.at[b],
                out_hbm.at[pl.ds(t0 * 1024, COUT)],
                sem_st.at[b],
            )

    for b in range(2):
        pltpu.make_async_copy(
            rows_v.at[b], out_hbm.at[pl.ds(0, COUT)], sem_st.at[b]
        ).wait()


@jax.jit
def _lookup(xt_flat, emb_flat):
    mesh = plsc.VectorSubcoreMesh(core_axis_name="c", subcore_axis_name="s")
    return pl.kernel(
        _lookup_body,
        out_type=jax.ShapeDtypeStruct((N * D,), jnp.float32),
        mesh=mesh,
        scratch_types=[
            pltpu.VMEM((V * D,), jnp.float32),
            pltpu.VMEM((2, CIDX), jnp.int32),
            pltpu.VMEM((2, COUT), jnp.float32),
            pltpu.SemaphoreType.DMA((2,)),
            pltpu.SemaphoreType.DMA((2,)),
        ],
        compiler_params=pltpu.CompilerParams(
            use_tc_tiling_on_sc=False, needs_layout_passes=False
        ),
    )(xt_flat, emb_flat)


def kernel(x, embedding):
    xt_flat = jnp.transpose(x).reshape(N).astype(jnp.int32)
    out = _lookup(xt_flat, embedding.reshape(V * D))
    out5 = out.reshape(S, D // 8, B // 128, 8, 128)
    return jnp.transpose(out5, (2, 4, 0, 1, 3)).reshape(B, S, D)


# speedup vs baseline: 4.0282x; 3.2017x over previous
import jax
import jax.numpy as jnp
from jax import lax
from jax.experimental import pallas as pl
from jax.experimental.pallas import tpu as pltpu
from jax.experimental.pallas import tpu_sc as plsc

B, S = 16384, 200
V, D = 50, 16
N = B * S
NW = 32
L = 16
NTILES = S * (D // 8) * (B // 128)
PER_W = NTILES // NW
TPC = 32
NCHUNKS = PER_W // TPC
CIDX = TPC * 128
GROUPS = CIDX // L
COUT = TPC * 1024


def _lookup_body(xt_hbm, emb_hbm, out_hbm, emb_v, idx_v, rows_v, sem_idx, sem_st):
    cid = lax.axis_index("c")
    sid = lax.axis_index("s")
    wid = sid * 2 + cid
    t_base = wid * PER_W

    pltpu.sync_copy(emb_hbm, emb_v)

    def start_idx(c, b):
        t0 = t_base + c * TPC
        s = t0 >> 8
        nt0 = t0 & 127
        pltpu.async_copy(
            xt_hbm.at[pl.ds(s * B + nt0 * 128, CIDX)], idx_v.at[b], sem_idx.at[b]
        )

    start_idx(0, 0)
    start_idx(1, 1)

    lane = lax.iota(jnp.int32, L)

    @pl.loop(0, NCHUNKS, step=2)
    def _super(g0):
        for b in range(2):
            c = g0 + b
            t0 = t_base + c * TPC
            dt = (t0 >> 7) & 1

            @pl.when(c >= 2)
            def _():
                pltpu.make_async_copy(
                    rows_v.at[b], out_hbm.at[pl.ds(0, COUT)], sem_st.at[b]
                ).wait()

            pltpu.make_async_copy(
                xt_hbm.at[pl.ds(0, CIDX)], idx_v.at[b], sem_idx.at[b]
            ).wait()

            idx_ref = idx_v.at[b]
            rows_ref = rows_v.at[b]

            @plsc.parallel_loop(0, GROUPS, unroll=4)
            def _grp(g):
                idxv = idx_ref[pl.ds(g * L, L)]
                gbase = (idxv << 8) + lane + (dt << 7)
                sbase = ((g >> 3) << 10) + ((g & 7) << 4)
                for di in range(8):
                    vals = plsc.load_gather(emb_v, [gbase + (di << 4)])
                    rows_ref[pl.ds(sbase + (di << 7), L)] = vals

            @pl.when(c + 2 < NCHUNKS)
            def _():
                start_idx(c + 2, b)

            pltpu.async_copy(
                rows_v.at[b],
                out_hbm.at[pl.ds(t0 * 1024, COUT)],
                sem_st.at[b],
            )

    for b in range(2):
        pltpu.make_async_copy(
            rows_v.at[b], out_hbm.at[pl.ds(0, COUT)], sem_st.at[b]
        ).wait()


@jax.jit
def _lookup(xt_flat, emb_flat):
    mesh = plsc.VectorSubcoreMesh(core_axis_name="c", subcore_axis_name="s")
    return pl.kernel(
        _lookup_body,
        out_type=jax.ShapeDtypeStruct((N * D,), jnp.float32),
        mesh=mesh,
        scratch_types=[
            pltpu.VMEM((V * D * L,), jnp.float32),
            pltpu.VMEM((2, CIDX), jnp.int32),
            pltpu.VMEM((2, COUT), jnp.float32),
            pltpu.SemaphoreType.DMA((2,)),
            pltpu.SemaphoreType.DMA((2,)),
        ],
        compiler_params=pltpu.CompilerParams(
            use_tc_tiling_on_sc=False, needs_layout_passes=False
        ),
    )(xt_flat, emb_flat)


def kernel(x, embedding):
    xt_flat = jnp.transpose(x).reshape(N).astype(jnp.int32)
    emb_skew = jnp.repeat(embedding.reshape(V * D), L)
    out = _lookup(xt_flat, emb_skew)
    out5 = out.reshape(S, D // 8, B // 128, 8, 128)
    return jnp.transpose(out5, (2, 4, 0, 1, 3)).reshape(B, S, D)


# speedup vs baseline: 4.2673x; 1.0594x over previous
import jax
import jax.numpy as jnp
from jax import lax
from jax.experimental import pallas as pl
from jax.experimental.pallas import tpu as pltpu
from jax.experimental.pallas import tpu_sc as plsc

B, S = 16384, 200
V, D = 50, 16
N = B * S
NW = 32
L = 16
NTILES = S * (D // 8) * (B // 128)
PER_W = NTILES // NW
TPC = 32
NCHUNKS = PER_W // TPC
CIDX = (TPC // 2) * 128
GROUPS = CIDX // L
HOUT = (TPC // 2) * 1024
COUT = TPC * 1024


def _lookup_body(xt_hbm, emb_hbm, out_hbm, emb_v, idx_v, rows_v, sem_idx, sem_st):
    cid = lax.axis_index("c")
    sid = lax.axis_index("s")
    wid = sid * 2 + cid
    u_base = wid * NCHUNKS

    pltpu.sync_copy(emb_hbm, emb_v)

    def start_idx(c, b):
        u = u_base + c
        s = u >> 3
        nt0 = (u & 7) << 4
        pltpu.async_copy(
            xt_hbm.at[pl.ds(s * B + nt0 * 128, CIDX)], idx_v.at[b], sem_idx.at[b]
        )

    start_idx(0, 0)
    start_idx(1, 1)

    lane = lax.iota(jnp.int32, L)

    @pl.loop(0, NCHUNKS, step=2)
    def _super(g0):
        for b in range(2):
            c = g0 + b
            u = u_base + c
            s = u >> 3
            nt0 = (u & 7) << 4

            @pl.when(c >= 2)
            def _():
                for dt in range(2):
                    pltpu.make_async_copy(
                        rows_v.at[b].at[pl.ds(dt * HOUT, HOUT)],
                        out_hbm.at[pl.ds(0, HOUT)],
                        sem_st.at[b],
                    ).wait()

            pltpu.make_async_copy(
                xt_hbm.at[pl.ds(0, CIDX)], idx_v.at[b], sem_idx.at[b]
            ).wait()

            idx_ref = idx_v.at[b]
            rows_ref = rows_v.at[b]

            @plsc.parallel_loop(0, GROUPS, unroll=4)
            def _grp(g):
                idxv = idx_ref[pl.ds(g * L, L)]
                gbase = (idxv << 8) + lane
                sbase = ((g >> 3) << 10) + ((g & 7) << 4)
                for d in range(D):
                    vals = plsc.load_gather(emb_v, [gbase + (d << 4)])
                    half = (d >> 3) * HOUT
                    rows_ref[pl.ds(half + sbase + ((d & 7) << 7), L)] = vals

            @pl.when(c + 2 < NCHUNKS)
            def _():
                start_idx(c + 2, b)

            for dt in range(2):
                pltpu.async_copy(
                    rows_v.at[b].at[pl.ds(dt * HOUT, HOUT)],
                    out_hbm.at[pl.ds(((s * 2 + dt) * 128 + nt0) * 1024, HOUT)],
                    sem_st.at[b],
                )

    for b in range(2):
        for dt in range(2):
            pltpu.make_async_copy(
                rows_v.at[b].at[pl.ds(dt * HOUT, HOUT)],
                out_hbm.at[pl.ds(0, HOUT)],
                sem_st.at[b],
            ).wait()


@jax.jit
def _lookup(xt_flat, emb_flat):
    mesh = plsc.VectorSubcoreMesh(core_axis_name="c", subcore_axis_name="s")
    return pl.kernel(
        _lookup_body,
        out_type=jax.ShapeDtypeStruct((N * D,), jnp.float32),
        mesh=mesh,
        scratch_types=[
            pltpu.VMEM((V * D * L,), jnp.float32),
            pltpu.VMEM((2, CIDX), jnp.int32),
            pltpu.VMEM((2, COUT), jnp.float32),
            pltpu.SemaphoreType.DMA((2,)),
            pltpu.SemaphoreType.DMA((2,)),
        ],
        compiler_params=pltpu.CompilerParams(
            use_tc_tiling_on_sc=False, needs_layout_passes=False
        ),
    )(xt_flat, emb_flat)


def kernel(x, embedding):
    xt_flat = jnp.transpose(x).reshape(N).astype(jnp.int32)
    emb_skew = jnp.repeat(embedding.reshape(V * D), L)
    out = _lookup(xt_flat, emb_skew)
    out5 = out.reshape(S, D // 8, B // 128, 8, 128)
    return jnp.transpose(out5, (2, 4, 0, 1, 3)).reshape(B, S, D)
